# baseline (speedup 1.0000x reference)
import functools

import jax
import jax.numpy as jnp
from jax import lax
from jax.experimental import pallas as pl
from jax.experimental.pallas import tpu as pltpu
from jax.experimental.pallas import tpu_sc as plsc

N = 10000
A = 2048
E = 320000
B = 4096
EMB = 10
H = 128

CH = 128
NCHUNK = E // CH
NTILE = 16
ROWS_I = 640
ROWS_I_LAST = N - 15 * ROWS_I
ROWS_A = A // NTILE
NPAD = 8
NCHUNK_P = 2560
E_P = NCHUNK_P * CH
KTILE = NCHUNK_P // NTILE
NRING = 4
NROUND = KTILE // NRING
AP = A + NPAD
NP_ = N + NPAD

_f32 = jnp.float32


def _feat_node_body(xt_ref, xf_ref, idt_ref, tabs_ref, fw_ref, out_ref):
    ids = xt_ref[...]
    rows = ids.shape[0]
    cols = [idt_ref[...]]
    for f in range(4):
        oh = (ids[:, f : f + 1]
              == lax.broadcasted_iota(jnp.int32, (rows, 100), 1)).astype(_f32)
        cols.append(jnp.dot(oh, tabs_ref[f], preferred_element_type=_f32))
    cols.append(jnp.dot(xf_ref[...], fw_ref[...], preferred_element_type=_f32))
    cols.append(jnp.zeros((rows, 4), _f32))
    out_ref[...] = jnp.concatenate(cols, axis=1)


def _feat_adm_body(xt_ref, xf_ref, tabs_ref, fw_ref, out_ref):
    ids = xt_ref[...]
    rows = ids.shape[0]
    cols = []
    for f in range(4):
        oh = (ids[:, f : f + 1]
              == lax.broadcasted_iota(jnp.int32, (rows, 100), 1)).astype(_f32)
        cols.append(jnp.dot(oh, tabs_ref[f], preferred_element_type=_f32))
    cols.append(jnp.dot(xf_ref[...], fw_ref[...], preferred_element_type=_f32))
    cols.append(jnp.zeros((rows, 14), _f32))
    out_ref[...] = jnp.concatenate(cols, axis=1)


def _pairtab_body(tabs_ref, out_ref):
    a = jnp.broadcast_to(tabs_ref[0][:, None, :], (100, 100, EMB))
    b = jnp.broadcast_to(tabs_ref[1][None, :, :], (100, 100, EMB))
    one = jnp.ones((100, 100, 1), _f32)
    z = jnp.zeros((100, 100, 11), _f32)
    out_ref[...] = jnp.concatenate([a, b, one, z], axis=2)


def _sc_edge_body(src_hbm, dst_hbm, tok_hbm, flt_hbm, feat_a_hbm, feat_n_hbm,
                  ptab_hbm, z64_hbm, z32_hbm,
                  segA_hbm, segP_hbm, segN_hbm, segPa_hbm,
                  srcb, dstb, tokb, pairb, pay64, pay32, fltb, accA, accP,
                  si0, si1, si2, si3, sg0, sg1, sg2, sg3, ss0, ss1, ss2, ss3):
    cid = lax.axis_index("c")
    sid = lax.axis_index("s")
    si = [si0, si1, si2, si3]
    sg = [sg0, sg1, sg2, sg3]
    ss = [ss0, ss1, ss2, ss3]

    @pl.when(jnp.logical_and(cid == 0, sid < 15))
    def _():
        @pl.loop(0, 5)
        def _(r):
            off = sid * ROWS_I + r * CH
            pltpu.sync_copy(z64_hbm, accA.at[pl.ds(off, CH)])
            pltpu.sync_copy(z32_hbm, accP.at[pl.ds(off, CH)])

    @pl.when(jnp.logical_and(cid == 0, sid == 15))
    def _():
        @pl.loop(0, 3)
        def _(r):
            off = 15 * ROWS_I + r * CH
            pltpu.sync_copy(z64_hbm, accA.at[pl.ds(off, CH)])
            pltpu.sync_copy(z32_hbm, accP.at[pl.ds(off, CH)])

        pltpu.sync_copy(z64_hbm.at[pl.ds(0, 16)], accA.at[pl.ds(9984, 16)])
        pltpu.sync_copy(z32_hbm.at[pl.ds(0, 16)], accP.at[pl.ds(9984, 16)])

    @pl.when(cid == 1)
    def _():
        off = sid * ROWS_A
        pltpu.sync_copy(z64_hbm, accA.at[pl.ds(off, CH)])
        pltpu.sync_copy(z32_hbm, accP.at[pl.ds(off, CH)])

    plsc.subcore_barrier()

    def load_descs(k, b):
        base = (sid + NTILE * k) * CH
        return [
            (src_hbm.at[pl.ds(base, CH)], srcb.at[b], si[b]),
            (dst_hbm.at[pl.ds(base, CH)], dstb.at[b], si[b]),
            (tok_hbm.at[pl.ds(2 * base, 2 * CH)], tokb.at[b], si[b]),
            (flt_hbm.at[pl.ds(4 * base, 4 * CH)], fltb.at[b], si[b]),
        ]

    def issue_loads(k, b):
        for sr, dr, sem in load_descs(k, b):
            pltpu.async_copy(sr, dr, sem)

    def wait_loads(k, b):
        for sr, dr, sem in load_descs(k, b):
            pltpu.make_async_copy(sr, dr, sem).wait()

    def compute_pair(b):
        idx0 = lax.iota(jnp.int32, 16) * 2
        for v in range(8):
            t0 = plsc.load_gather(tokb.at[b], [idx0 + 32 * v])
            t1 = plsc.load_gather(tokb.at[b], [idx0 + 32 * v + 1])
            pairb[b, pl.ds(16 * v, 16)] = t0 * 100 + t1

    def inject_flt(b):
        l = lax.iota(jnp.int32, 16)
        row0 = lax.shift_right_logical(l, 2)
        col = 24 + lax.bitwise_and(l, 3)
        for v in range(32):
            fv = fltb[b, pl.ds(16 * v, 16)]
            plsc.store_scatter(pay32.at[b], [row0 + 4 * v, col], fv)

    def issue_gathers(k, b):
        @pl.when(cid == 0)
        def _():
            pltpu.async_copy(feat_a_hbm.at[srcb.at[b]], pay64.at[b], sg[b])

        @pl.when(cid == 1)
        def _():
            pltpu.async_copy(feat_n_hbm.at[dstb.at[b]], pay64.at[b], sg[b])

        pltpu.async_copy(ptab_hbm.at[pairb.at[b]], pay32.at[b], sg[b])

    def wait_gathers(k, b):
        @pl.when(cid == 0)
        def _():
            pltpu.make_async_copy(feat_a_hbm.at[srcb.at[b]], pay64.at[b],
                                  sg[b]).wait()

        @pl.when(cid == 1)
        def _():
            pltpu.make_async_copy(feat_n_hbm.at[dstb.at[b]], pay64.at[b],
                                  sg[b]).wait()

        pltpu.make_async_copy(ptab_hbm.at[pairb.at[b]], pay32.at[b],
                              sg[b]).wait()

    def issue_scats(k, b):
        @pl.when(cid == 0)
        def _():
            pltpu.async_copy(pay64.at[b], accA.at[dstb.at[b]], ss[b], add=True)
            pltpu.async_copy(pay32.at[b], accP.at[dstb.at[b]], ss[b], add=True)

        @pl.when(cid == 1)
        def _():
            pltpu.async_copy(pay64.at[b], accA.at[srcb.at[b]], ss[b], add=True)
            pltpu.async_copy(pay32.at[b], accP.at[srcb.at[b]], ss[b], add=True)

    def wait_scats(k, b):
        @pl.when(cid == 0)
        def _():
            pltpu.make_async_copy(pay64.at[b], accA.at[dstb.at[b]], ss[b]).wait()
            pltpu.make_async_copy(pay32.at[b], accP.at[dstb.at[b]], ss[b]).wait()

        @pl.when(cid == 1)
        def _():
            pltpu.make_async_copy(pay64.at[b], accA.at[srcb.at[b]], ss[b]).wait()
            pltpu.make_async_copy(pay32.at[b], accP.at[srcb.at[b]], ss[b]).wait()

    issue_loads(0, 0)
    issue_loads(1, 1)
    wait_loads(0, 0)
    compute_pair(0)
    issue_gathers(0, 0)

    @pl.loop(0, NROUND)
    def _(j):
        for b in range(NRING):
            k = NRING * j + b
            b1 = (b + 1) % NRING
            b2 = (b + 2) % NRING

            @pl.when(k + 2 < KTILE)
            def _(k=k, b2=b2):
                @pl.when(k >= 2)
                def _():
                    wait_scats(k - 2, b2)

                issue_loads(k + 2, b2)

            @pl.when(k + 1 < KTILE)
            def _(k=k, b1=b1):
                wait_loads(k + 1, b1)
                compute_pair(b1)
                issue_gathers(k + 1, b1)

            wait_gathers(k, b)
            inject_flt(b)
            issue_scats(k, b)

    for ktail in range(KTILE - NRING, KTILE):
        wait_scats(ktail, ktail % NRING)

    plsc.subcore_barrier()

    @pl.when(jnp.logical_and(cid == 0, sid < 15))
    def _():
        sl = pl.ds(sid * ROWS_I, ROWS_I)
        pltpu.sync_copy(accA.at[sl], segA_hbm.at[sl])
        pltpu.sync_copy(accP.at[sl], segP_hbm.at[sl])

    @pl.when(jnp.logical_and(cid == 0, sid == 15))
    def _():
        sl = pl.ds(15 * ROWS_I, ROWS_I_LAST)
        pltpu.sync_copy(accA.at[sl], segA_hbm.at[sl])
        pltpu.sync_copy(accP.at[sl], segP_hbm.at[sl])

    @pl.when(cid == 1)
    def _():
        sl = pl.ds(sid * ROWS_A, ROWS_A)
        pltpu.sync_copy(accA.at[sl], segN_hbm.at[sl])
        pltpu.sync_copy(accP.at[sl], segPa_hbm.at[sl])


def _combine_body(segA_ref, segP_ref, segN_ref, segPa_ref,
                  featn_ref, feata_ref, nW_ref, nb_ref, aW_ref, ab_ref,
                  eW_ref, eb_ref, efW_ref, wsi_ref, wni_ref, wsa_ref, wna_ref,
                  item_out_ref, adm_out_ref):
    nW = nW_ref[...]
    aW = aW_ref[...]
    eW = eW_ref[...]
    nb = nb_ref[...]
    ab = ab_ref[...]
    eb = eb_ref[...]
    We20 = eW[0:20, :]
    Wf_fold = jnp.dot(efW_ref[...], eW[20:30, :], preferred_element_type=_f32)

    item_h = jnp.dot(featn_ref[...][:, :60], nW, preferred_element_type=_f32) + nb
    adm_h = jnp.dot(feata_ref[...][:, :50], aW, preferred_element_type=_f32) + ab

    segP = segP_ref[...]
    cnt_i = segP[:, 20:21]
    seg_i = (jnp.dot(segA_ref[...][:, :50], aW, preferred_element_type=_f32)
             + jnp.dot(segP[:, :20], We20, preferred_element_type=_f32)
             + jnp.dot(segP[:, 24:28], Wf_fold, preferred_element_type=_f32)
             + cnt_i * (ab + eb))
    agg_i = seg_i / jnp.maximum(cnt_i, 1.0)
    item_out_ref[...] = jax.nn.relu(
        jnp.dot(item_h, wsi_ref[...], preferred_element_type=_f32)
        + jnp.dot(agg_i, wni_ref[...], preferred_element_type=_f32))

    segPa = segPa_ref[...]
    cnt_a = segPa[:, 20:21]
    seg_a = (jnp.dot(segN_ref[...][:, :60], nW, preferred_element_type=_f32)
             + jnp.dot(segPa[:, :20], We20, preferred_element_type=_f32)
             + jnp.dot(segPa[:, 24:28], Wf_fold, preferred_element_type=_f32)
             + cnt_a * (nb + eb))
    agg_a = seg_a / jnp.maximum(cnt_a, 1.0)
    adm_out_ref[...] = jax.nn.relu(
        jnp.dot(adm_h, wsa_ref[...], preferred_element_type=_f32)
        + jnp.dot(agg_a, wna_ref[...], preferred_element_type=_f32))


def _sc_gather_body(qa_hbm, qi_hbm, adm_hbm, item_hbm, ga_hbm, gi_hbm,
                    qb, gbuf):
    cid = lax.axis_index("c")
    sid = lax.axis_index("s")
    wid = sid * 2 + cid
    sl = pl.ds(wid * CH, CH)
    pltpu.sync_copy(qa_hbm.at[sl], qb)
    pltpu.sync_copy(adm_hbm.at[qb], gbuf)
    pltpu.sync_copy(gbuf, ga_hbm.at[sl])
    pltpu.sync_copy(qi_hbm.at[sl], qb)
    pltpu.sync_copy(item_hbm.at[qb], gbuf)
    pltpu.sync_copy(gbuf, gi_hbm.at[sl])


def _dot_body(ga_ref, gi_ref, out_ref):
    out_ref[...] = jnp.sum(ga_ref[...] * gi_ref[...], axis=1, keepdims=True)


def kernel(node_id, x_tok, x_flt, adm_x_tok, adm_x_flt, edge_src, edge_dst,
           edge_tok, edge_flt, q_adm, q_item, item_id_table, node_tok_tables,
           node_flt_W, adm_tok_tables, adm_flt_W, edge_tok_tables, edge_flt_W,
           node_align_W, node_align_b, adm_align_W, adm_align_b, edge_align_W,
           edge_align_b, W_self_item, W_nbr_item, W_self_adm, W_nbr_adm):
    feat_n = pl.pallas_call(
        _feat_node_body,
        grid=(5,),
        in_specs=[
            pl.BlockSpec((N // 5, 4), lambda i: (i, 0)),
            pl.BlockSpec((N // 5, 8), lambda i: (i, 0)),
            pl.BlockSpec((N // 5, EMB), lambda i: (i, 0)),
            pl.BlockSpec((4, 100, EMB), lambda i: (0, 0, 0)),
            pl.BlockSpec((8, EMB), lambda i: (0, 0)),
        ],
        out_specs=pl.BlockSpec((N // 5, 64), lambda i: (i, 0)),
        out_shape=jax.ShapeDtypeStruct((N, 64), _f32),
    )(x_tok, x_flt, item_id_table, node_tok_tables, node_flt_W)

    feat_a = pl.pallas_call(
        _feat_adm_body,
        in_specs=[
            pl.BlockSpec((A, 4), lambda: (0, 0)),
            pl.BlockSpec((A, 8), lambda: (0, 0)),
            pl.BlockSpec((4, 100, EMB), lambda: (0, 0, 0)),
            pl.BlockSpec((8, EMB), lambda: (0, 0)),
        ],
        out_specs=pl.BlockSpec((A, 64), lambda: (0, 0)),
        out_shape=jax.ShapeDtypeStruct((A, 64), _f32),
    )(adm_x_tok, adm_x_flt, adm_tok_tables, adm_flt_W)

    pair_tab = pl.pallas_call(
        _pairtab_body,
        in_specs=[pl.BlockSpec((2, 100, EMB), lambda: (0, 0, 0))],
        out_specs=pl.BlockSpec((100, 100, 32), lambda: (0, 0, 0)),
        out_shape=jax.ShapeDtypeStruct((100, 100, 32), _f32),
    )(edge_tok_tables).reshape(10000, 32)

    padlen = E_P - E
    padmod = jnp.arange(padlen, dtype=jnp.int32) % NPAD
    edge_src_p = jnp.concatenate([edge_src, A + padmod])
    edge_dst_p = jnp.concatenate([edge_dst, N + padmod])
    tok_flat_p = jnp.concatenate(
        [edge_tok.reshape(E * 2), jnp.zeros((padlen * 2,), jnp.int32)])
    flt_p = jnp.concatenate(
        [edge_flt.reshape(E * 4), jnp.zeros((padlen * 4,), _f32)])
    feat_a_p = jnp.pad(feat_a, ((0, NPAD), (0, 0)))
    feat_n_p = jnp.pad(feat_n, ((0, NPAD), (0, 0)))
    ptab_p = jnp.pad(pair_tab, ((0, NPAD), (0, 0)))
    z64s = jnp.zeros((CH, 64), _f32)
    z32s = jnp.zeros((CH, 32), _f32)

    sc_edge = functools.partial(
        pl.kernel,
        out_type=[
            jax.ShapeDtypeStruct((N, 64), _f32),
            jax.ShapeDtypeStruct((N, 32), _f32),
            jax.ShapeDtypeStruct((A, 64), _f32),
            jax.ShapeDtypeStruct((A, 32), _f32),
        ],
        mesh=plsc.VectorSubcoreMesh(core_axis_name="c", subcore_axis_name="s"),
        compiler_params=pltpu.CompilerParams(use_tc_tiling_on_sc=False,
                                             needs_layout_passes=False),
        scratch_types=[
            pltpu.VMEM((NRING, CH), jnp.int32),
            pltpu.VMEM((NRING, CH), jnp.int32),
            pltpu.VMEM((NRING, 2 * CH), jnp.int32),
            pltpu.VMEM((NRING, CH), jnp.int32),
            pltpu.VMEM((NRING, CH, 64), _f32),
            pltpu.VMEM((NRING, CH, 32), _f32),
            pltpu.VMEM((NRING, 4 * CH), _f32),
            pltpu.VMEM_SHARED((NP_, 64), _f32),
            pltpu.VMEM_SHARED((NP_, 32), _f32),
        ] + [pltpu.SemaphoreType.DMA] * 12,
    )(_sc_edge_body)
    segA, segP, segN, segPa = sc_edge(
        edge_src_p, edge_dst_p, tok_flat_p, flt_p, feat_a_p, feat_n_p,
        ptab_p, z64s, z32s)

    item_out, adm_out = pl.pallas_call(
        _combine_body,
        in_specs=[
            pl.BlockSpec((N, 64), lambda: (0, 0)),
            pl.BlockSpec((N, 32), lambda: (0, 0)),
            pl.BlockSpec((A, 64), lambda: (0, 0)),
            pl.BlockSpec((A, 32), lambda: (0, 0)),
            pl.BlockSpec((N, 64), lambda: (0, 0)),
            pl.BlockSpec((A, 64), lambda: (0, 0)),
            pl.BlockSpec((60, H), lambda: (0, 0)),
            pl.BlockSpec((1, H), lambda: (0, 0)),
            pl.BlockSpec((50, H), lambda: (0, 0)),
            pl.BlockSpec((1, H), lambda: (0, 0)),
            pl.BlockSpec((30, H), lambda: (0, 0)),
            pl.BlockSpec((1, H), lambda: (0, 0)),
            pl.BlockSpec((4, EMB), lambda: (0, 0)),
            pl.BlockSpec((H, H), lambda: (0, 0)),
            pl.BlockSpec((H, H), lambda: (0, 0)),
            pl.BlockSpec((H, H), lambda: (0, 0)),
            pl.BlockSpec((H, H), lambda: (0, 0)),
        ],
        out_specs=[
            pl.BlockSpec((N, H), lambda: (0, 0)),
            pl.BlockSpec((A, H), lambda: (0, 0)),
        ],
        out_shape=[
            jax.ShapeDtypeStruct((N, H), _f32),
            jax.ShapeDtypeStruct((A, H), _f32),
        ],
    )(segA, segP, segN, segPa, feat_n, feat_a,
      node_align_W, node_align_b.reshape(1, H), adm_align_W,
      adm_align_b.reshape(1, H), edge_align_W, edge_align_b.reshape(1, H),
      edge_flt_W, W_self_item, W_nbr_item, W_self_adm, W_nbr_adm)

    sc_gather = functools.partial(
        pl.kernel,
        out_type=[
            jax.ShapeDtypeStruct((B, H), _f32),
            jax.ShapeDtypeStruct((B, H), _f32),
        ],
        mesh=plsc.VectorSubcoreMesh(core_axis_name="c", subcore_axis_name="s"),
        compiler_params=pltpu.CompilerParams(use_tc_tiling_on_sc=False),
        scratch_types=[
            pltpu.VMEM((CH,), jnp.int32),
            pltpu.VMEM((CH, H), _f32),
        ],
    )(_sc_gather_body)
    ga, gi = sc_gather(q_adm, q_item, adm_out, item_out)

    logits = pl.pallas_call(
        _dot_body,
        in_specs=[
            pl.BlockSpec((B, H), lambda: (0, 0)),
            pl.BlockSpec((B, H), lambda: (0, 0)),
        ],
        out_specs=pl.BlockSpec((B, 1), lambda: (0, 0)),
        out_shape=jax.ShapeDtypeStruct((B, 1), _f32),
    )(ga, gi)
    return logits.reshape(B)


# speedup vs baseline: 1.1026x; 1.1026x over previous
import functools

import jax
import jax.numpy as jnp
from jax import lax
from jax.experimental import pallas as pl
from jax.experimental.pallas import tpu as pltpu
from jax.experimental.pallas import tpu_sc as plsc

N = 10000
A = 2048
E = 320000
B = 4096
EMB = 10
H = 128

CH = 128
NCHUNK = E // CH
NTILE = 16
ROWS_I = 640
ROWS_I_LAST = N - 15 * ROWS_I
ROWS_A = A // NTILE
NPAD = 8
NCHUNK_P = 2560
E_P = NCHUNK_P * CH
KTILE = NCHUNK_P // NTILE
NRING = 4
NROUND = KTILE // NRING
AP = A + NPAD
NP_ = N + NPAD

_f32 = jnp.float32


def _feat_node_body(xt_ref, xf_ref, idt_ref, tabs_ref, fw_ref, out_ref):
    ids = xt_ref[...]
    rows = ids.shape[0]
    cols = [idt_ref[...]]
    for f in range(4):
        oh = (ids[:, f : f + 1]
              == lax.broadcasted_iota(jnp.int32, (rows, 100), 1)).astype(_f32)
        cols.append(jnp.dot(oh, tabs_ref[f], preferred_element_type=_f32))
    cols.append(jnp.dot(xf_ref[...], fw_ref[...], preferred_element_type=_f32))
    cols.append(jnp.zeros((rows, 4), _f32))
    out_ref[...] = jnp.concatenate(cols, axis=1)


def _feat_adm_body(xt_ref, xf_ref, tabs_ref, fw_ref, out_ref):
    ids = xt_ref[...]
    rows = ids.shape[0]
    cols = []
    for f in range(4):
        oh = (ids[:, f : f + 1]
              == lax.broadcasted_iota(jnp.int32, (rows, 100), 1)).astype(_f32)
        cols.append(jnp.dot(oh, tabs_ref[f], preferred_element_type=_f32))
    cols.append(jnp.dot(xf_ref[...], fw_ref[...], preferred_element_type=_f32))
    cols.append(jnp.zeros((rows, 14), _f32))
    out_ref[...] = jnp.concatenate(cols, axis=1)


def _pairtab_body(tabs_ref, out_ref):
    a = jnp.broadcast_to(tabs_ref[0][:, None, :], (100, 100, EMB))
    b = jnp.broadcast_to(tabs_ref[1][None, :, :], (100, 100, EMB))
    one = jnp.ones((100, 100, 1), _f32)
    z = jnp.zeros((100, 100, 11), _f32)
    out_ref[...] = jnp.concatenate([a, b, one, z], axis=2)


EP_BLK = E_P // 32
EP_CHK = NCHUNK_P // 32


def _edgeprep_body(tok_ref, flt_ref, pair_ref, fltr_ref):
    pid = pl.program_id(0)
    t = tok_ref[...]
    eid = pid * EP_BLK + lax.broadcasted_iota(jnp.int32, (EP_BLK, 1), 0)
    valid = eid < E
    pairv = jnp.where(valid, t[:, 0:1] * 100 + t[:, 1:2], 10000)
    pair_ref[...] = pairv.reshape(EP_CHK, CH)
    f = jnp.where(valid, flt_ref[...], 0.0)
    fltr_ref[...] = jnp.transpose(f.reshape(EP_CHK, CH, 4), (0, 2, 1))


def _sc_edge_body(src_hbm, dst_hbm, pair_hbm, flt_hbm, feat_a_hbm, feat_n_hbm,
                  ptab_hbm, z64_hbm, z32_hbm,
                  segA_hbm, segP_hbm, segN_hbm, segPa_hbm,
                  srcb, dstb, pairb, pay64, pay32, fltb, accA, accP,
                  si0, si1, si2, si3, sg0, sg1, sg2, sg3, ss0, ss1, ss2, ss3):
    cid = lax.axis_index("c")
    sid = lax.axis_index("s")
    si = [si0, si1, si2, si3]
    sg = [sg0, sg1, sg2, sg3]
    ss = [ss0, ss1, ss2, ss3]

    @pl.when(jnp.logical_and(cid == 0, sid < 15))
    def _():
        @pl.loop(0, 5)
        def _(r):
            off = sid * ROWS_I + r * CH
            pltpu.sync_copy(z64_hbm, accA.at[pl.ds(off, CH)])
            pltpu.sync_copy(z32_hbm, accP.at[pl.ds(off, CH)])

    @pl.when(jnp.logical_and(cid == 0, sid == 15))
    def _():
        @pl.loop(0, 3)
        def _(r):
            off = 15 * ROWS_I + r * CH
            pltpu.sync_copy(z64_hbm, accA.at[pl.ds(off, CH)])
            pltpu.sync_copy(z32_hbm, accP.at[pl.ds(off, CH)])

        pltpu.sync_copy(z64_hbm.at[pl.ds(0, 16)], accA.at[pl.ds(9984, 16)])
        pltpu.sync_copy(z32_hbm.at[pl.ds(0, 16)], accP.at[pl.ds(9984, 16)])

    @pl.when(cid == 1)
    def _():
        off = sid * ROWS_A
        pltpu.sync_copy(z64_hbm, accA.at[pl.ds(off, CH)])
        pltpu.sync_copy(z32_hbm, accP.at[pl.ds(off, CH)])

    plsc.subcore_barrier()

    def load_descs(k, b):
        base = (sid + NTILE * k) * CH
        return [
            (src_hbm.at[pl.ds(base, CH)], srcb.at[b], si[b]),
            (dst_hbm.at[pl.ds(base, CH)], dstb.at[b], si[b]),
            (pair_hbm.at[pl.ds(base, CH)], pairb.at[b], si[b]),
            (flt_hbm.at[pl.ds(4 * base, 4 * CH)], fltb.at[b], si[b]),
        ]

    def issue_loads(k, b):
        for sr, dr, sem in load_descs(k, b):
            pltpu.async_copy(sr, dr, sem)

    def wait_loads(k, b):
        for sr, dr, sem in load_descs(k, b):
            pltpu.make_async_copy(sr, dr, sem).wait()

    def inject_flt(b):
        l = lax.iota(jnp.int32, 16)
        for v in range(32):
            fv = fltb[b, pl.ds(16 * v, 16)]
            rowv = l + 16 * (v % 8)
            colv = jnp.full((16,), 24 + v // 8, jnp.int32)
            plsc.store_scatter(pay32.at[b], [rowv, colv], fv)

    def issue_gathers(k, b):
        @pl.when(cid == 0)
        def _():
            pltpu.async_copy(feat_a_hbm.at[srcb.at[b]], pay64.at[b], sg[b])

        @pl.when(cid == 1)
        def _():
            pltpu.async_copy(feat_n_hbm.at[dstb.at[b]], pay64.at[b], sg[b])

        pltpu.async_copy(ptab_hbm.at[pairb.at[b]], pay32.at[b], sg[b])

    def wait_gathers(k, b):
        @pl.when(cid == 0)
        def _():
            pltpu.make_async_copy(feat_a_hbm.at[srcb.at[b]], pay64.at[b],
                                  sg[b]).wait()

        @pl.when(cid == 1)
        def _():
            pltpu.make_async_copy(feat_n_hbm.at[dstb.at[b]], pay64.at[b],
                                  sg[b]).wait()

        pltpu.make_async_copy(ptab_hbm.at[pairb.at[b]], pay32.at[b],
                              sg[b]).wait()

    def issue_scats(k, b):
        @pl.when(cid == 0)
        def _():
            pltpu.async_copy(pay64.at[b], accA.at[dstb.at[b]], ss[b], add=True)
            pltpu.async_copy(pay32.at[b], accP.at[dstb.at[b]], ss[b], add=True)

        @pl.when(cid == 1)
        def _():
            pltpu.async_copy(pay64.at[b], accA.at[srcb.at[b]], ss[b], add=True)
            pltpu.async_copy(pay32.at[b], accP.at[srcb.at[b]], ss[b], add=True)

    def wait_scats(k, b):
        @pl.when(cid == 0)
        def _():
            pltpu.make_async_copy(pay64.at[b], accA.at[dstb.at[b]], ss[b]).wait()
            pltpu.make_async_copy(pay32.at[b], accP.at[dstb.at[b]], ss[b]).wait()

        @pl.when(cid == 1)
        def _():
            pltpu.make_async_copy(pay64.at[b], accA.at[srcb.at[b]], ss[b]).wait()
            pltpu.make_async_copy(pay32.at[b], accP.at[srcb.at[b]], ss[b]).wait()

    issue_loads(0, 0)
    issue_loads(1, 1)
    wait_loads(0, 0)
    issue_gathers(0, 0)

    @pl.loop(0, NROUND)
    def _(j):
        for b in range(NRING):
            k = NRING * j + b
            b1 = (b + 1) % NRING
            b2 = (b + 2) % NRING

            @pl.when(k + 2 < KTILE)
            def _(k=k, b2=b2):
                @pl.when(k >= 2)
                def _():
                    wait_scats(k - 2, b2)

                issue_loads(k + 2, b2)

            @pl.when(k + 1 < KTILE)
            def _(k=k, b1=b1):
                wait_loads(k + 1, b1)
                issue_gathers(k + 1, b1)

            wait_gathers(k, b)
            inject_flt(b)
            issue_scats(k, b)

    for ktail in range(KTILE - NRING, KTILE):
        wait_scats(ktail, ktail % NRING)

    plsc.subcore_barrier()

    @pl.when(jnp.logical_and(cid == 0, sid < 15))
    def _():
        sl = pl.ds(sid * ROWS_I, ROWS_I)
        pltpu.sync_copy(accA.at[sl], segA_hbm.at[sl])
        pltpu.sync_copy(accP.at[sl], segP_hbm.at[sl])

    @pl.when(jnp.logical_and(cid == 0, sid == 15))
    def _():
        sl = pl.ds(15 * ROWS_I, ROWS_I_LAST)
        pltpu.sync_copy(accA.at[sl], segA_hbm.at[sl])
        pltpu.sync_copy(accP.at[sl], segP_hbm.at[sl])

    @pl.when(cid == 1)
    def _():
        sl = pl.ds(sid * ROWS_A, ROWS_A)
        pltpu.sync_copy(accA.at[sl], segN_hbm.at[sl])
        pltpu.sync_copy(accP.at[sl], segPa_hbm.at[sl])


def _combine_body(segA_ref, segP_ref, segN_ref, segPa_ref,
                  featn_ref, feata_ref, nW_ref, nb_ref, aW_ref, ab_ref,
                  eW_ref, eb_ref, efW_ref, wsi_ref, wni_ref, wsa_ref, wna_ref,
                  item_out_ref, adm_out_ref):
    nW = nW_ref[...]
    aW = aW_ref[...]
    eW = eW_ref[...]
    nb = nb_ref[...]
    ab = ab_ref[...]
    eb = eb_ref[...]
    We20 = eW[0:20, :]
    Wf_fold = jnp.dot(efW_ref[...], eW[20:30, :], preferred_element_type=_f32)

    item_h = jnp.dot(featn_ref[...][:, :60], nW, preferred_element_type=_f32) + nb
    adm_h = jnp.dot(feata_ref[...][:, :50], aW, preferred_element_type=_f32) + ab

    segP = segP_ref[...]
    cnt_i = segP[:, 20:21]
    seg_i = (jnp.dot(segA_ref[...][:, :50], aW, preferred_element_type=_f32)
             + jnp.dot(segP[:, :20], We20, preferred_element_type=_f32)
             + jnp.dot(segP[:, 24:28], Wf_fold, preferred_element_type=_f32)
             + cnt_i * (ab + eb))
    agg_i = seg_i / jnp.maximum(cnt_i, 1.0)
    item_out_ref[...] = jax.nn.relu(
        jnp.dot(item_h, wsi_ref[...], preferred_element_type=_f32)
        + jnp.dot(agg_i, wni_ref[...], preferred_element_type=_f32))

    segPa = segPa_ref[...]
    cnt_a = segPa[:, 20:21]
    seg_a = (jnp.dot(segN_ref[...][:, :60], nW, preferred_element_type=_f32)
             + jnp.dot(segPa[:, :20], We20, preferred_element_type=_f32)
             + jnp.dot(segPa[:, 24:28], Wf_fold, preferred_element_type=_f32)
             + cnt_a * (nb + eb))
    agg_a = seg_a / jnp.maximum(cnt_a, 1.0)
    adm_out_ref[...] = jax.nn.relu(
        jnp.dot(adm_h, wsa_ref[...], preferred_element_type=_f32)
        + jnp.dot(agg_a, wna_ref[...], preferred_element_type=_f32))


def _sc_gather_body(qa_hbm, qi_hbm, adm_hbm, item_hbm, ga_hbm, gi_hbm,
                    qb, gbuf):
    cid = lax.axis_index("c")
    sid = lax.axis_index("s")
    wid = sid * 2 + cid
    sl = pl.ds(wid * CH, CH)
    pltpu.sync_copy(qa_hbm.at[sl], qb)
    pltpu.sync_copy(adm_hbm.at[qb], gbuf)
    pltpu.sync_copy(gbuf, ga_hbm.at[sl])
    pltpu.sync_copy(qi_hbm.at[sl], qb)
    pltpu.sync_copy(item_hbm.at[qb], gbuf)
    pltpu.sync_copy(gbuf, gi_hbm.at[sl])


def _dot_body(ga_ref, gi_ref, out_ref):
    out_ref[...] = jnp.sum(ga_ref[...] * gi_ref[...], axis=1, keepdims=True)


def kernel(node_id, x_tok, x_flt, adm_x_tok, adm_x_flt, edge_src, edge_dst,
           edge_tok, edge_flt, q_adm, q_item, item_id_table, node_tok_tables,
           node_flt_W, adm_tok_tables, adm_flt_W, edge_tok_tables, edge_flt_W,
           node_align_W, node_align_b, adm_align_W, adm_align_b, edge_align_W,
           edge_align_b, W_self_item, W_nbr_item, W_self_adm, W_nbr_adm):
    feat_n = pl.pallas_call(
        _feat_node_body,
        grid=(5,),
        in_specs=[
            pl.BlockSpec((N // 5, 4), lambda i: (i, 0)),
            pl.BlockSpec((N // 5, 8), lambda i: (i, 0)),
            pl.BlockSpec((N // 5, EMB), lambda i: (i, 0)),
            pl.BlockSpec((4, 100, EMB), lambda i: (0, 0, 0)),
            pl.BlockSpec((8, EMB), lambda i: (0, 0)),
        ],
        out_specs=pl.BlockSpec((N // 5, 64), lambda i: (i, 0)),
        out_shape=jax.ShapeDtypeStruct((N, 64), _f32),
    )(x_tok, x_flt, item_id_table, node_tok_tables, node_flt_W)

    feat_a = pl.pallas_call(
        _feat_adm_body,
        in_specs=[
            pl.BlockSpec((A, 4), lambda: (0, 0)),
            pl.BlockSpec((A, 8), lambda: (0, 0)),
            pl.BlockSpec((4, 100, EMB), lambda: (0, 0, 0)),
            pl.BlockSpec((8, EMB), lambda: (0, 0)),
        ],
        out_specs=pl.BlockSpec((A, 64), lambda: (0, 0)),
        out_shape=jax.ShapeDtypeStruct((A, 64), _f32),
    )(adm_x_tok, adm_x_flt, adm_tok_tables, adm_flt_W)

    pair_tab = pl.pallas_call(
        _pairtab_body,
        in_specs=[pl.BlockSpec((2, 100, EMB), lambda: (0, 0, 0))],
        out_specs=pl.BlockSpec((100, 100, 32), lambda: (0, 0, 0)),
        out_shape=jax.ShapeDtypeStruct((100, 100, 32), _f32),
    )(edge_tok_tables).reshape(10000, 32)

    padlen = E_P - E
    padmod = jnp.arange(padlen, dtype=jnp.int32) % NPAD
    edge_src_p = jnp.concatenate([edge_src, A + padmod])
    edge_dst_p = jnp.concatenate([edge_dst, N + padmod])
    pair2d, flt2d = pl.pallas_call(
        _edgeprep_body,
        grid=(32,),
        in_specs=[
            pl.BlockSpec((EP_BLK, 2), lambda i: (i, 0)),
            pl.BlockSpec((EP_BLK, 4), lambda i: (i, 0)),
        ],
        out_specs=[
            pl.BlockSpec((EP_CHK, CH), lambda i: (i, 0)),
            pl.BlockSpec((EP_CHK, 4, CH), lambda i: (i, 0, 0)),
        ],
        out_shape=[
            jax.ShapeDtypeStruct((NCHUNK_P, CH), jnp.int32),
            jax.ShapeDtypeStruct((NCHUNK_P, 4, CH), _f32),
        ],
    )(edge_tok, edge_flt)
    pair_flat_p = pair2d.reshape(E_P)
    flt_flat_p = flt2d.reshape(E_P * 4)
    feat_a_p = jnp.pad(feat_a, ((0, NPAD), (0, 0)))
    feat_n_p = jnp.pad(feat_n, ((0, NPAD), (0, 0)))
    ptab_p = jnp.pad(pair_tab, ((0, NPAD), (0, 0)))
    z64s = jnp.zeros((CH, 64), _f32)
    z32s = jnp.zeros((CH, 32), _f32)

    sc_edge = functools.partial(
        pl.kernel,
        out_type=[
            jax.ShapeDtypeStruct((N, 64), _f32),
            jax.ShapeDtypeStruct((N, 32), _f32),
            jax.ShapeDtypeStruct((A, 64), _f32),
            jax.ShapeDtypeStruct((A, 32), _f32),
        ],
        mesh=plsc.VectorSubcoreMesh(core_axis_name="c", subcore_axis_name="s"),
        compiler_params=pltpu.CompilerParams(use_tc_tiling_on_sc=False,
                                             needs_layout_passes=False),
        scratch_types=[
            pltpu.VMEM((NRING, CH), jnp.int32),
            pltpu.VMEM((NRING, CH), jnp.int32),
            pltpu.VMEM((NRING, CH), jnp.int32),
            pltpu.VMEM((NRING, CH, 64), _f32),
            pltpu.VMEM((NRING, CH, 32), _f32),
            pltpu.VMEM((NRING, 4 * CH), _f32),
            pltpu.VMEM_SHARED((NP_, 64), _f32),
            pltpu.VMEM_SHARED((NP_, 32), _f32),
        ] + [pltpu.SemaphoreType.DMA] * 12,
    )(_sc_edge_body)
    segA, segP, segN, segPa = sc_edge(
        edge_src_p, edge_dst_p, pair_flat_p, flt_flat_p, feat_a_p, feat_n_p,
        ptab_p, z64s, z32s)

    item_out, adm_out = pl.pallas_call(
        _combine_body,
        in_specs=[
            pl.BlockSpec((N, 64), lambda: (0, 0)),
            pl.BlockSpec((N, 32), lambda: (0, 0)),
            pl.BlockSpec((A, 64), lambda: (0, 0)),
            pl.BlockSpec((A, 32), lambda: (0, 0)),
            pl.BlockSpec((N, 64), lambda: (0, 0)),
            pl.BlockSpec((A, 64), lambda: (0, 0)),
            pl.BlockSpec((60, H), lambda: (0, 0)),
            pl.BlockSpec((1, H), lambda: (0, 0)),
            pl.BlockSpec((50, H), lambda: (0, 0)),
            pl.BlockSpec((1, H), lambda: (0, 0)),
            pl.BlockSpec((30, H), lambda: (0, 0)),
            pl.BlockSpec((1, H), lambda: (0, 0)),
            pl.BlockSpec((4, EMB), lambda: (0, 0)),
            pl.BlockSpec((H, H), lambda: (0, 0)),
            pl.BlockSpec((H, H), lambda: (0, 0)),
            pl.BlockSpec((H, H), lambda: (0, 0)),
            pl.BlockSpec((H, H), lambda: (0, 0)),
        ],
        out_specs=[
            pl.BlockSpec((N, H), lambda: (0, 0)),
            pl.BlockSpec((A, H), lambda: (0, 0)),
        ],
        out_shape=[
            jax.ShapeDtypeStruct((N, H), _f32),
            jax.ShapeDtypeStruct((A, H), _f32),
        ],
    )(segA, segP, segN, segPa, feat_n, feat_a,
      node_align_W, node_align_b.reshape(1, H), adm_align_W,
      adm_align_b.reshape(1, H), edge_align_W, edge_align_b.reshape(1, H),
      edge_flt_W, W_self_item, W_nbr_item, W_self_adm, W_nbr_adm)

    sc_gather = functools.partial(
        pl.kernel,
        out_type=[
            jax.ShapeDtypeStruct((B, H), _f32),
            jax.ShapeDtypeStruct((B, H), _f32),
        ],
        mesh=plsc.VectorSubcoreMesh(core_axis_name="c", subcore_axis_name="s"),
        compiler_params=pltpu.CompilerParams(use_tc_tiling_on_sc=False),
        scratch_types=[
            pltpu.VMEM((CH,), jnp.int32),
            pltpu.VMEM((CH, H), _f32),
        ],
    )(_sc_gather_body)
    ga, gi = sc_gather(q_adm, q_item, adm_out, item_out)

    logits = pl.pallas_call(
        _dot_body,
        in_specs=[
            pl.BlockSpec((B, H), lambda: (0, 0)),
            pl.BlockSpec((B, H), lambda: (0, 0)),
        ],
        out_specs=pl.BlockSpec((B, 1), lambda: (0, 0)),
        out_shape=jax.ShapeDtypeStruct((B, 1), _f32),
    )(ga, gi)
    return logits.reshape(B)
